# baseline (speedup 1.0000x reference)
import functools

import jax
import jax.numpy as jnp
from jax import lax
from jax.experimental import pallas as pl
from jax.experimental.pallas import tpu as pltpu
from jax.experimental.pallas import tpu_sc as plsc

_B, _S = 4, 2048
_N = _B * _S
_HIDDEN = 768
_DDIM = 128
_NW = 32
_WCHUNK = 64
_EPS = 1e-12

_SLICE_ROWS = (1, 3)

_TOK_BLK = 2048
_S_BLKS = _S // _TOK_BLK


def _sc_gather_slice(word_ids, domain_ids, word_emb, domain_emb, off, ntok):
    mesh = plsc.VectorSubcoreMesh(core_axis_name="c", subcore_axis_name="s")
    tpw = ntok // _NW
    wchunk = min(_WCHUNK, tpw // 2)
    nw = tpw // wchunk
    dchunk = tpw // 2

    @functools.partial(
        pl.kernel,
        out_type=[
            jax.ShapeDtypeStruct((ntok, _HIDDEN), jnp.float32),
            jax.ShapeDtypeStruct((ntok, _DDIM), jnp.float32),
        ],
        mesh=mesh,
        scratch_types=[
            pltpu.VMEM((tpw,), jnp.int32),
            pltpu.VMEM((tpw,), jnp.int32),
            pltpu.VMEM((wchunk, _HIDDEN), jnp.float32),
            pltpu.VMEM((wchunk, _HIDDEN), jnp.float32),
            pltpu.VMEM((dchunk, _DDIM), jnp.float32),
            pltpu.SemaphoreType.DMA,
            pltpu.SemaphoreType.DMA,
            pltpu.SemaphoreType.DMA,
            pltpu.SemaphoreType.DMA,
            pltpu.SemaphoreType.DMA,
            pltpu.SemaphoreType.DMA,
            pltpu.SemaphoreType.DMA,
        ],
    )
    def k(wids_hbm, dids_hbm, wtab_hbm, dtab_hbm, ow_hbm, od_hbm,
          widx_v, didx_v, wb0, wb1, db, sI, sW0, sW1, sC0, sC1, sD, sCD):
        wid = lax.axis_index("s") * 2 + lax.axis_index("c")
        obase = wid * tpw
        base = off + obase
        wb = (wb0, wb1)
        sW = (sW0, sW1)
        sC = (sC0, sC1)
        ci0 = pltpu.async_copy(wids_hbm.at[pl.ds(base, tpw)], widx_v, sI)
        ci1 = pltpu.async_copy(dids_hbm.at[pl.ds(base, tpw)], didx_v, sI)
        ci0.wait()
        ci1.wait()
        g = [None] * nw
        co = [None] * nw
        g[0] = pltpu.async_copy(
            wtab_hbm.at[widx_v.at[pl.ds(0, wchunk)]], wb[0], sW[0])
        g[1] = pltpu.async_copy(
            wtab_hbm.at[widx_v.at[pl.ds(wchunk, wchunk)]], wb[1], sW[1])
        gd = pltpu.async_copy(dtab_hbm.at[didx_v.at[pl.ds(0, dchunk)]], db, sD)
        g[0].wait()
        co[0] = pltpu.async_copy(wb[0], ow_hbm.at[pl.ds(obase, wchunk)], sC[0])
        g[1].wait()
        co[1] = pltpu.async_copy(
            wb[1], ow_hbm.at[pl.ds(obase + wchunk, wchunk)], sC[1])
        for c in range(2, nw):
            co[c - 2].wait()
            g[c] = pltpu.async_copy(
                wtab_hbm.at[widx_v.at[pl.ds(c * wchunk, wchunk)]],
                wb[c % 2], sW[c % 2])
        gd.wait()
        cd0 = pltpu.async_copy(db, od_hbm.at[pl.ds(obase, dchunk)], sCD)
        for c in range(2, nw):
            g[c].wait()
            co[c] = pltpu.async_copy(
                wb[c % 2], ow_hbm.at[pl.ds(obase + c * wchunk, wchunk)],
                sC[c % 2])
        cd0.wait()
        gd1 = pltpu.async_copy(
            dtab_hbm.at[didx_v.at[pl.ds(dchunk, dchunk)]], db, sD)
        gd1.wait()
        cd1 = pltpu.async_copy(db, od_hbm.at[pl.ds(obase + dchunk, dchunk)],
                               sCD)
        for c in range(max(nw - 2, 0), nw):
            co[c].wait()
        cd1.wait()

    return k(word_ids, domain_ids, word_emb, domain_emb)


def _tc_body(gw_ref, gd_ref, pos_ref, cr_ref, w_ref, g_ref, bb_ref, o_ref):
    proj = lax.dot_general(
        gd_ref[...], w_ref[...], (((1,), (0,)), ((), ())),
        precision=lax.Precision.DEFAULT, preferred_element_type=jnp.float32)
    e = gw_ref[...] + proj + pos_ref[...] + cr_ref[0, :][None, :]
    mean = jnp.mean(e, axis=1, keepdims=True)
    c = e - mean
    var = jnp.mean(c * c, axis=1, keepdims=True)
    o_ref[...] = (c * lax.rsqrt(var + _EPS)) * g_ref[0, :][None, :] \
        + bb_ref[0, :][None, :]


def _tc_body_alias(gw_ref, gd_ref, pos_ref, cr_ref, w_ref, g_ref, bb_ref,
                   prev_ref, o_ref):
    _tc_body(gw_ref, gd_ref, pos_ref, cr_ref, w_ref, g_ref, bb_ref, o_ref)


def _tc_fuse_slice(gw, gd, pos_emb, const_row, W_proj, ln_gamma, ln_beta,
                   boff, nrows, prev=None):
    in_specs = [
        pl.BlockSpec((_TOK_BLK, _HIDDEN), lambda i, j: (j * _S_BLKS + i, 0)),
        pl.BlockSpec((_TOK_BLK, _DDIM), lambda i, j: (j * _S_BLKS + i, 0)),
        pl.BlockSpec((_TOK_BLK, _HIDDEN), lambda i, j: (i, 0)),
        pl.BlockSpec((1, _HIDDEN), lambda i, j: (0, 0)),
        pl.BlockSpec((_DDIM, _HIDDEN), lambda i, j: (0, 0)),
        pl.BlockSpec((1, _HIDDEN), lambda i, j: (0, 0)),
        pl.BlockSpec((1, _HIDDEN), lambda i, j: (0, 0)),
    ]
    args = [gw, gd, pos_emb, const_row, W_proj, ln_gamma, ln_beta]
    kwargs = {}
    body = _tc_body
    if prev is not None:
        in_specs.append(pl.BlockSpec(memory_space=pl.ANY))
        args.append(prev)
        kwargs["input_output_aliases"] = {7: 0}
        body = _tc_body_alias
    return pl.pallas_call(
        body,
        grid=(_S_BLKS, nrows),
        in_specs=in_specs,
        out_specs=pl.BlockSpec(
            (_TOK_BLK, _HIDDEN),
            lambda i, j: ((boff + j) * _S_BLKS + i, 0)),
        out_shape=jax.ShapeDtypeStruct((_N, _HIDDEN), jnp.float32),
        **kwargs,
    )(*args)


def kernel(input_ids, domain_ids, word_emb, domain_emb, pos_emb, type_emb,
           W_proj, b_proj, ln_gamma, ln_beta):
    wids = input_ids.reshape(-1).astype(jnp.int32)
    dids = domain_ids.reshape(-1).astype(jnp.int32)
    const_row = (type_emb[0] + b_proj)[None, :]
    gamma = ln_gamma[None, :]
    beta = ln_beta[None, :]
    gs = []
    row0 = 0
    for nrows in _SLICE_ROWS:
        gs.append(_sc_gather_slice(wids, dids, word_emb, domain_emb,
                                   row0 * _S, nrows * _S))
        row0 += nrows
    out = None
    row0 = 0
    for (gw, gd), nrows in zip(gs, _SLICE_ROWS):
        out = _tc_fuse_slice(gw, gd, pos_emb, const_row, W_proj, gamma, beta,
                             boff=row0, nrows=nrows, prev=out)
        row0 += nrows
    return out.reshape(_B, _S, _HIDDEN)


# speedup vs baseline: 1.0959x; 1.0959x over previous
import functools

import jax
import jax.numpy as jnp
from jax import lax
from jax.experimental import pallas as pl
from jax.experimental.pallas import tpu as pltpu
from jax.experimental.pallas import tpu_sc as plsc

_B, _S = 4, 2048
_N = _B * _S
_HIDDEN = 768
_DDIM = 128
_NW = 32
_WCHUNK = 64
_EPS = 1e-12

_SLICE_ROWS = (2, 2)

_TOK_BLK = 2048
_S_BLKS = _S // _TOK_BLK


def _sc_gather_slice(word_ids, domain_ids, word_emb, domain_emb, off, ntok):
    mesh = plsc.VectorSubcoreMesh(core_axis_name="c", subcore_axis_name="s")
    tpw = ntok // _NW
    wchunk = min(_WCHUNK, tpw // 2)
    nw = tpw // wchunk
    dchunk = tpw // 2

    @functools.partial(
        pl.kernel,
        out_type=[
            jax.ShapeDtypeStruct((ntok, _HIDDEN), jnp.float32),
            jax.ShapeDtypeStruct((ntok, _DDIM), jnp.float32),
        ],
        mesh=mesh,
        scratch_types=[
            pltpu.VMEM((tpw,), jnp.int32),
            pltpu.VMEM((tpw,), jnp.int32),
            pltpu.VMEM((wchunk, _HIDDEN), jnp.float32),
            pltpu.VMEM((wchunk, _HIDDEN), jnp.float32),
            pltpu.VMEM((dchunk, _DDIM), jnp.float32),
            pltpu.SemaphoreType.DMA,
            pltpu.SemaphoreType.DMA,
            pltpu.SemaphoreType.DMA,
            pltpu.SemaphoreType.DMA,
            pltpu.SemaphoreType.DMA,
            pltpu.SemaphoreType.DMA,
            pltpu.SemaphoreType.DMA,
        ],
    )
    def k(wids_hbm, dids_hbm, wtab_hbm, dtab_hbm, ow_hbm, od_hbm,
          widx_v, didx_v, wb0, wb1, db, sI, sW0, sW1, sC0, sC1, sD, sCD):
        wid = lax.axis_index("s") * 2 + lax.axis_index("c")
        obase = wid * tpw
        base = off + obase
        wb = (wb0, wb1)
        sW = (sW0, sW1)
        sC = (sC0, sC1)
        ci0 = pltpu.async_copy(wids_hbm.at[pl.ds(base, tpw)], widx_v, sI)
        ci1 = pltpu.async_copy(dids_hbm.at[pl.ds(base, tpw)], didx_v, sI)
        ci0.wait()
        ci1.wait()
        g = [None] * nw
        co = [None] * nw
        g[0] = pltpu.async_copy(
            wtab_hbm.at[widx_v.at[pl.ds(0, wchunk)]], wb[0], sW[0])
        g[1] = pltpu.async_copy(
            wtab_hbm.at[widx_v.at[pl.ds(wchunk, wchunk)]], wb[1], sW[1])
        gd = pltpu.async_copy(dtab_hbm.at[didx_v.at[pl.ds(0, dchunk)]], db, sD)
        g[0].wait()
        co[0] = pltpu.async_copy(wb[0], ow_hbm.at[pl.ds(obase, wchunk)], sC[0])
        g[1].wait()
        co[1] = pltpu.async_copy(
            wb[1], ow_hbm.at[pl.ds(obase + wchunk, wchunk)], sC[1])
        for c in range(2, nw):
            co[c - 2].wait()
            g[c] = pltpu.async_copy(
                wtab_hbm.at[widx_v.at[pl.ds(c * wchunk, wchunk)]],
                wb[c % 2], sW[c % 2])
        gd.wait()
        cd0 = pltpu.async_copy(db, od_hbm.at[pl.ds(obase, dchunk)], sCD)
        for c in range(2, nw):
            g[c].wait()
            co[c] = pltpu.async_copy(
                wb[c % 2], ow_hbm.at[pl.ds(obase + c * wchunk, wchunk)],
                sC[c % 2])
        cd0.wait()
        gd1 = pltpu.async_copy(
            dtab_hbm.at[didx_v.at[pl.ds(dchunk, dchunk)]], db, sD)
        gd1.wait()
        cd1 = pltpu.async_copy(db, od_hbm.at[pl.ds(obase + dchunk, dchunk)],
                               sCD)
        for c in range(max(nw - 2, 0), nw):
            co[c].wait()
        cd1.wait()

    return k(word_ids, domain_ids, word_emb, domain_emb)


def _tc_body(gw_ref, gd_ref, pos_ref, cr_ref, w_ref, g_ref, bb_ref, o_ref):
    proj = lax.dot_general(
        gd_ref[...], w_ref[...], (((1,), (0,)), ((), ())),
        precision=lax.Precision.DEFAULT, preferred_element_type=jnp.float32)
    e = gw_ref[...] + proj + pos_ref[...] + cr_ref[0, :][None, :]
    mean = jnp.mean(e, axis=1, keepdims=True)
    c = e - mean
    var = jnp.mean(c * c, axis=1, keepdims=True)
    o_ref[...] = (c * lax.rsqrt(var + _EPS)) * g_ref[0, :][None, :] \
        + bb_ref[0, :][None, :]


def _tc_body_alias(gw_ref, gd_ref, pos_ref, cr_ref, w_ref, g_ref, bb_ref,
                   prev_ref, o_ref):
    _tc_body(gw_ref, gd_ref, pos_ref, cr_ref, w_ref, g_ref, bb_ref, o_ref)


def _tc_fuse_slice(gw, gd, pos_emb, const_row, W_proj, ln_gamma, ln_beta,
                   boff, nrows, prev=None):
    in_specs = [
        pl.BlockSpec((_TOK_BLK, _HIDDEN), lambda i, j: (j * _S_BLKS + i, 0)),
        pl.BlockSpec((_TOK_BLK, _DDIM), lambda i, j: (j * _S_BLKS + i, 0)),
        pl.BlockSpec((_TOK_BLK, _HIDDEN), lambda i, j: (i, 0)),
        pl.BlockSpec((1, _HIDDEN), lambda i, j: (0, 0)),
        pl.BlockSpec((_DDIM, _HIDDEN), lambda i, j: (0, 0)),
        pl.BlockSpec((1, _HIDDEN), lambda i, j: (0, 0)),
        pl.BlockSpec((1, _HIDDEN), lambda i, j: (0, 0)),
    ]
    args = [gw, gd, pos_emb, const_row, W_proj, ln_gamma, ln_beta]
    kwargs = {}
    body = _tc_body
    if prev is not None:
        in_specs.append(pl.BlockSpec(memory_space=pl.ANY))
        args.append(prev)
        kwargs["input_output_aliases"] = {7: 0}
        body = _tc_body_alias
    return pl.pallas_call(
        body,
        grid=(_S_BLKS, nrows),
        in_specs=in_specs,
        out_specs=pl.BlockSpec(
            (_TOK_BLK, _HIDDEN),
            lambda i, j: ((boff + j) * _S_BLKS + i, 0)),
        out_shape=jax.ShapeDtypeStruct((_N, _HIDDEN), jnp.float32),
        **kwargs,
    )(*args)


def kernel(input_ids, domain_ids, word_emb, domain_emb, pos_emb, type_emb,
           W_proj, b_proj, ln_gamma, ln_beta):
    wids = input_ids.reshape(-1).astype(jnp.int32)
    dids = domain_ids.reshape(-1).astype(jnp.int32)
    const_row = (type_emb[0] + b_proj)[None, :]
    gamma = ln_gamma[None, :]
    beta = ln_beta[None, :]
    gs = []
    row0 = 0
    for nrows in _SLICE_ROWS:
        gs.append(_sc_gather_slice(wids, dids, word_emb, domain_emb,
                                   row0 * _S, nrows * _S))
        row0 += nrows
    out = None
    row0 = 0
    for (gw, gd), nrows in zip(gs, _SLICE_ROWS):
        out = _tc_fuse_slice(gw, gd, pos_emb, const_row, W_proj, gamma, beta,
                             boff=row0, nrows=nrows, prev=out)
        row0 += nrows
    return out.reshape(_B, _S, _HIDDEN)
